# Initial kernel scaffold; baseline (speedup 1.0000x reference)
import jax
import jax.numpy as jnp
from jax.experimental import pallas as pl


def kernel(x, stem_w, stem_b, st0_s2_wdw1, st0_s2_bdw1, st0_s2_wp1, st0_s2_bp1, st0_s2_w21, st0_s2_b21, st0_s2_wdw2, st0_s2_bdw2, st0_s2_w22, st0_s2_b22, st0_s1_w1, st0_s1_b1, st0_s1_wdw, st0_s1_bdw, st0_s1_w2, st0_s1_b2, st0_s1_mask, st0_out_perm, st1_s2_wdw1, st1_s2_bdw1, st1_s2_wp1, st1_s2_bp1, st1_s2_w21, st1_s2_b21, st1_s2_wdw2, st1_s2_bdw2, st1_s2_w22, st1_s2_b22, st1_s1_w1, st1_s1_b1, st1_s1_wdw, st1_s1_bdw, st1_s1_w2, st1_s1_b2, st1_s1_mask, st1_out_perm, st2_s2_wdw1, st2_s2_bdw1, st2_s2_wp1, st2_s2_bp1, st2_s2_w21, st2_s2_b21, st2_s2_wdw2, st2_s2_bdw2, st2_s2_w22, st2_s2_b22, st2_s1_w1, st2_s1_b1, st2_s1_wdw, st2_s1_bdw, st2_s1_w2, st2_s1_b2, st2_s1_mask, st2_out_perm):
    raise NotImplementedError("write your pallas kernel here")


# speedup vs baseline: 1.0990x; 1.0990x over previous
import functools

import jax
import jax.numpy as jnp
from jax import lax
from jax.experimental import pallas as pl
from jax.experimental.pallas import tpu as pltpu

_F32 = jnp.float32
_BF16 = jnp.bfloat16


def _shift_down(a, k):
    if k == 0:
        return a
    z = jnp.zeros((a.shape[0], k, a.shape[2]), a.dtype)
    return jnp.concatenate([z, a[:, :-k, :]], axis=1)


def _shift_up(a, k):
    if k == 0:
        return a
    z = jnp.zeros((a.shape[0], k, a.shape[2]), a.dtype)
    return jnp.concatenate([a[:, k:, :], z], axis=1)


def _col_masks(R, W):
    col = lax.broadcasted_iota(jnp.int32, (1, R, 1), 1) % W
    m_left = (col > 0).astype(_F32)
    m_right = (col < W - 1).astype(_F32)
    return m_left, m_right


def _dw3x3_s1(h1, wdw, bias, W, mL, mR):
    w = [wdw[k] for k in range(9)]
    center = bias + h1 * w[4]
    center = center + _shift_down(h1, W) * w[1] + _shift_up(h1, W) * w[7]
    left = (_shift_down(h1, W + 1) * w[0] + _shift_down(h1, 1) * w[3]
            + _shift_up(h1, W - 1) * w[6])
    right = (_shift_down(h1, W - 1) * w[2] + _shift_up(h1, 1) * w[5]
             + _shift_up(h1, W + 1) * w[8])
    return center + mL * left + mR * right


def _dw3x3_s2(z00, z01, z10, z11, wdw, bias, Wo, mL):
    w = [wdw[k] for k in range(9)]
    d = bias + z00 * w[4] + z01 * w[5] + z10 * w[7] + z11 * w[8]
    d = d + _shift_down(z10, Wo) * w[1] + _shift_down(z11, Wo) * w[2]
    left = (_shift_down(z01, 1) * w[3] + _shift_down(z11, 1) * w[6]
            + _shift_down(z11, Wo + 1) * w[0])
    return d + mL * left


def _mm(a3, w, b):
    B, R, K = a3.shape
    y = jnp.dot(a3.reshape(B * R, K).astype(_BF16), w,
                preferred_element_type=_F32) + b
    return y.reshape(B, R, w.shape[1])


def _stage_kernel(xpar_ref, wdw1_ref, bdw1_ref, wp1_ref, bp1_ref,
                  w21_ref, b21_ref, wdw2_ref, bdw2_ref, w22_ref, b22_ref,
                  w1_ref, b1_ref, wdw_ref, bdw_ref, w2_ref, b2_ref, mask_ref,
                  o_ref, *, nb, Ho, Wo):
    B = xpar_ref.shape[0]
    Rh = Ho * Wo
    cin = xpar_ref.shape[-1]
    bf = wp1_ref.shape[-1]
    mL, mR = _col_masks(Rh, Wo)

    xp = xpar_ref[...]
    h = jnp.maximum(
        jnp.dot(xp.reshape(B * 4 * Rh, cin), w21_ref[...],
                preferred_element_type=_F32) + b21_ref[...], 0.0)
    h = h.reshape(B, 2, 2, Rh, bf)

    z00 = xp[:, 0, 0].astype(_F32)
    z01 = xp[:, 0, 1].astype(_F32)
    z10 = xp[:, 1, 0].astype(_F32)
    z11 = xp[:, 1, 1].astype(_F32)
    d1 = _dw3x3_s2(z00, z01, z10, z11, wdw1_ref, bdw1_ref[...], Wo, mL)
    z1 = jnp.maximum(_mm(d1, wp1_ref[...], bp1_ref[...]), 0.0)

    d2 = _dw3x3_s2(h[:, 0, 0], h[:, 0, 1], h[:, 1, 0], h[:, 1, 1],
                   wdw2_ref, bdw2_ref[...], Wo, mL)
    z2 = jnp.maximum(_mm(d2, w22_ref[...], b22_ref[...]), 0.0)

    x = jnp.concatenate([z1, z2], axis=-1)

    for blk in range(nb):
        h1 = jnp.maximum(_mm(x, w1_ref[blk], b1_ref[blk]), 0.0)
        dd = _dw3x3_s1(h1, wdw_ref[blk], bdw_ref[blk], Wo, mL, mR)
        y = jnp.maximum(_mm(dd, w2_ref[blk], b2_ref[blk]), 0.0)
        x = jnp.where(mask_ref[blk] > 0.5, x, y)
    o_ref[...] = x.astype(o_ref.dtype)


def _stage_call(xpar, wdw1, bdw1, wp1, bp1, w21, b21, wdw2, bdw2, w22, b22,
                w1, b1, wdw, bdw, w2, b2, mask, Ho, Wo):
    n = xpar.shape[0]
    Rh = Ho * Wo
    cin = xpar.shape[-1]
    bf = wp1.shape[-1]
    nb = w1.shape[0]
    c = 2 * bf
    B = 1
    for cand in (8, 6, 4, 3, 2):
        if n % cand == 0:
            B = cand
            break
    body = functools.partial(_stage_kernel, nb=nb, Ho=Ho, Wo=Wo)
    weights = (wdw1, bdw1, wp1, bp1, w21, b21, wdw2, bdw2, w22, b22,
               w1, b1, wdw, bdw, w2, b2, mask)

    def _full(a):
        nd = a.ndim
        return pl.BlockSpec(a.shape, lambda i, _nd=nd: (0,) * _nd)

    flops = n * (2 * 4 * Rh * cin * bf + 2 * Rh * cin * bf + 2 * Rh * bf * bf
                 + 18 * Rh * (cin + bf)
                 + nb * (2 * Rh * c * bf + 2 * Rh * bf * c + 18 * Rh * bf))
    wbytes = sum(int(v.size) * v.dtype.itemsize for v in weights)
    cost = pl.CostEstimate(flops=int(flops), transcendentals=0,
                           bytes_accessed=int(xpar.size * 2 + n * Rh * c * 2
                                              + wbytes))
    return pl.pallas_call(
        body,
        out_shape=jax.ShapeDtypeStruct((n, Rh, c), _BF16),
        grid=(n // B,),
        in_specs=[pl.BlockSpec((B, 2, 2, Rh, cin),
                               lambda i: (i, 0, 0, 0, 0))]
                 + [_full(v) for v in weights],
        out_specs=pl.BlockSpec((B, Rh, c), lambda i: (i, 0, 0)),
        compiler_params=pltpu.CompilerParams(
            dimension_semantics=("parallel",)),
        cost_estimate=cost,
    )(xpar, *weights)


def _stem_kernel(x_ref, w_ref, b_ref, o_ref):
    y = jnp.dot(x_ref[...], w_ref[...], preferred_element_type=_F32)
    o_ref[...] = jnp.maximum(y + b_ref[...], 0.0).astype(o_ref.dtype)


def _stem_matmul(cols, w, b):
    rows, k = cols.shape
    cout = w.shape[1]
    tm = rows
    for cand in (4096, 2048, 1024, 512, 256, 128, 64, 32, 16, 8):
        if rows % cand == 0:
            tm = cand
            break
    cost = pl.CostEstimate(
        flops=int(2 * rows * k * cout), transcendentals=0,
        bytes_accessed=int(cols.size * 2 + w.size * 2 + rows * cout * 2))
    return pl.pallas_call(
        _stem_kernel,
        out_shape=jax.ShapeDtypeStruct((rows, cout), _BF16),
        grid=(rows // tm,),
        in_specs=[
            pl.BlockSpec((tm, k), lambda i: (i, 0)),
            pl.BlockSpec((k, cout), lambda i: (0, 0)),
            pl.BlockSpec((1, cout), lambda i: (0, 0)),
        ],
        out_specs=pl.BlockSpec((tm, cout), lambda i: (i, 0)),
        compiler_params=pltpu.CompilerParams(
            dimension_semantics=("parallel",)),
        cost_estimate=cost,
    )(cols, w, b)


def _parity_split(x_nhwc):
    n, h, w, c = x_nhwc.shape
    hh, wh = h // 2, w // 2
    x = x_nhwc.reshape(n, hh, 2, wh, 2, c)
    x = jnp.transpose(x, (0, 2, 4, 1, 3, 5))
    return x.reshape(n, 2, 2, hh * wh, c)


def kernel(x, stem_w, stem_b, st0_s2_wdw1, st0_s2_bdw1, st0_s2_wp1,
           st0_s2_bp1, st0_s2_w21, st0_s2_b21, st0_s2_wdw2, st0_s2_bdw2,
           st0_s2_w22, st0_s2_b22, st0_s1_w1, st0_s1_b1, st0_s1_wdw,
           st0_s1_bdw, st0_s1_w2, st0_s1_b2, st0_s1_mask, st0_out_perm,
           st1_s2_wdw1, st1_s2_bdw1, st1_s2_wp1, st1_s2_bp1, st1_s2_w21,
           st1_s2_b21, st1_s2_wdw2, st1_s2_bdw2, st1_s2_w22, st1_s2_b22,
           st1_s1_w1, st1_s1_b1, st1_s1_wdw, st1_s1_bdw, st1_s1_w2,
           st1_s1_b2, st1_s1_mask, st1_out_perm, st2_s2_wdw1, st2_s2_bdw1,
           st2_s2_wp1, st2_s2_bp1, st2_s2_w21, st2_s2_b21, st2_s2_wdw2,
           st2_s2_bdw2, st2_s2_w22, st2_s2_b22, st2_s1_w1, st2_s1_b1,
           st2_s1_wdw, st2_s1_bdw, st2_s1_w2, st2_s1_b2, st2_s1_mask,
           st2_out_perm):
    xh = jnp.transpose(x, (0, 2, 3, 1)).astype(_BF16)
    n, h, w, _ = xh.shape
    ho, wo = (h - 1) // 2 + 1, (w - 1) // 2 + 1
    xp = jnp.pad(xh, ((0, 0), (1, 1), (1, 1), (0, 0)))
    cols = jnp.concatenate(
        [xp[:, ky:ky + 2 * ho - 1:2, kx:kx + 2 * wo - 1:2, :]
         .reshape(n * ho * wo, 3) for ky in range(3) for kx in range(3)],
        axis=-1)
    c1 = stem_w.shape[1]
    feat = _stem_matmul(cols, stem_w, stem_b).reshape(n, ho, wo, c1)
    feat = lax.reduce_window(feat, jnp.array(-jnp.inf, feat.dtype), lax.max,
                             (1, 3, 3, 1), (1, 2, 2, 1),
                             ((0, 0), (1, 1), (1, 1), (0, 0)))
    hh, ww = feat.shape[1], feat.shape[2]

    stages = [
        (st0_s2_wdw1, st0_s2_bdw1, st0_s2_wp1, st0_s2_bp1, st0_s2_w21,
         st0_s2_b21, st0_s2_wdw2, st0_s2_bdw2, st0_s2_w22, st0_s2_b22,
         st0_s1_w1, st0_s1_b1, st0_s1_wdw, st0_s1_bdw, st0_s1_w2, st0_s1_b2,
         st0_s1_mask, st0_out_perm),
        (st1_s2_wdw1, st1_s2_bdw1, st1_s2_wp1, st1_s2_bp1, st1_s2_w21,
         st1_s2_b21, st1_s2_wdw2, st1_s2_bdw2, st1_s2_w22, st1_s2_b22,
         st1_s1_w1, st1_s1_b1, st1_s1_wdw, st1_s1_bdw, st1_s1_w2, st1_s1_b2,
         st1_s1_mask, st1_out_perm),
        (st2_s2_wdw1, st2_s2_bdw1, st2_s2_wp1, st2_s2_bp1, st2_s2_w21,
         st2_s2_b21, st2_s2_wdw2, st2_s2_bdw2, st2_s2_w22, st2_s2_b22,
         st2_s1_w1, st2_s1_b1, st2_s1_wdw, st2_s1_bdw, st2_s1_w2, st2_s1_b2,
         st2_s1_mask, st2_out_perm),
    ]
    outs = []
    for sp in stages:
        xpar = _parity_split(feat)
        hh, ww = hh // 2, ww // 2
        xflat = _stage_call(xpar, *sp[:17], Ho=hh, Wo=ww)
        c = xflat.shape[-1]
        feat = xflat.reshape(n, hh, ww, c)
        xo = jnp.take(feat, sp[17], axis=-1)
        outs.append(jnp.transpose(xo, (0, 3, 1, 2)).astype(_F32))
    return tuple(outs)


# speedup vs baseline: 6.4470x; 5.8663x over previous
import functools

import jax
import jax.numpy as jnp
from jax import lax
from jax.experimental import pallas as pl
from jax.experimental.pallas import tpu as pltpu

_F32 = jnp.float32
_BF16 = jnp.bfloat16


def _shift_down(a, k):
    if k == 0:
        return a
    z = jnp.zeros((a.shape[0], k, a.shape[2]), a.dtype)
    return jnp.concatenate([z, a[:, :-k, :]], axis=1)


def _shift_up(a, k):
    if k == 0:
        return a
    z = jnp.zeros((a.shape[0], k, a.shape[2]), a.dtype)
    return jnp.concatenate([a[:, k:, :], z], axis=1)


def _col_masks(R, W):
    col = lax.broadcasted_iota(jnp.int32, (1, R, 1), 1) % W
    m_left = (col > 0).astype(_F32)
    m_right = (col < W - 1).astype(_F32)
    return m_left, m_right


def _dw3x3_s1(h1, wdw, bias, W, mL, mR):
    w = [wdw[k] for k in range(9)]
    center = bias + h1 * w[4]
    center = center + _shift_down(h1, W) * w[1] + _shift_up(h1, W) * w[7]
    left = (_shift_down(h1, W + 1) * w[0] + _shift_down(h1, 1) * w[3]
            + _shift_up(h1, W - 1) * w[6])
    right = (_shift_down(h1, W - 1) * w[2] + _shift_up(h1, 1) * w[5]
             + _shift_up(h1, W + 1) * w[8])
    return center + mL * left + mR * right


def _dw3x3_s2(z00, z01, z10, z11, wdw, bias, Wo, mL):
    w = [wdw[k] for k in range(9)]
    d = bias + z00 * w[4] + z01 * w[5] + z10 * w[7] + z11 * w[8]
    d = d + _shift_down(z10, Wo) * w[1] + _shift_down(z11, Wo) * w[2]
    left = (_shift_down(z01, 1) * w[3] + _shift_down(z11, 1) * w[6]
            + _shift_down(z11, Wo + 1) * w[0])
    return d + mL * left


def _mm(a3, w, b):
    B, R, K = a3.shape
    y = jnp.dot(a3.reshape(B * R, K).astype(_BF16), w,
                preferred_element_type=_F32) + b
    return y.reshape(B, R, w.shape[1])


def _stage_kernel(xpar_ref, wdw1_ref, bdw1_ref, wp1_ref, bp1_ref,
                  w21_ref, b21_ref, wdw2_ref, bdw2_ref, w22_ref, b22_ref,
                  w1_ref, b1_ref, wdw_ref, bdw_ref, w2_ref, b2_ref, mask_ref,
                  o_ref, *, nb, Ho, Wo):
    B = xpar_ref.shape[0]
    Rh = Ho * Wo
    cin = xpar_ref.shape[-1]
    bf = wp1_ref.shape[-1]
    mL, mR = _col_masks(Rh, Wo)

    xp = xpar_ref[...]
    h = jnp.maximum(
        jnp.dot(xp.reshape(B * 4 * Rh, cin), w21_ref[...],
                preferred_element_type=_F32) + b21_ref[...], 0.0)
    h = h.reshape(B, 2, 2, Rh, bf)

    z00 = xp[:, 0, 0].astype(_F32)
    z01 = xp[:, 0, 1].astype(_F32)
    z10 = xp[:, 1, 0].astype(_F32)
    z11 = xp[:, 1, 1].astype(_F32)
    d1 = _dw3x3_s2(z00, z01, z10, z11, wdw1_ref, bdw1_ref[...], Wo, mL)
    z1 = jnp.maximum(_mm(d1, wp1_ref[...], bp1_ref[...]), 0.0)

    d2 = _dw3x3_s2(h[:, 0, 0], h[:, 0, 1], h[:, 1, 0], h[:, 1, 1],
                   wdw2_ref, bdw2_ref[...], Wo, mL)
    z2 = jnp.maximum(_mm(d2, w22_ref[...], b22_ref[...]), 0.0)

    x = jnp.concatenate([z1, z2], axis=-1)

    for blk in range(nb):
        h1 = jnp.maximum(_mm(x, w1_ref[blk], b1_ref[blk]), 0.0)
        dd = _dw3x3_s1(h1, wdw_ref[blk], bdw_ref[blk], Wo, mL, mR)
        y = jnp.maximum(_mm(dd, w2_ref[blk], b2_ref[blk]), 0.0)
        x = jnp.where(mask_ref[blk] > 0.5, x, y)
    o_ref[...] = x.astype(o_ref.dtype)


def _stage_call(xpar, wdw1, bdw1, wp1, bp1, w21, b21, wdw2, bdw2, w22, b22,
                w1, b1, wdw, bdw, w2, b2, mask, Ho, Wo):
    n = xpar.shape[0]
    Rh = Ho * Wo
    cin = xpar.shape[-1]
    bf = wp1.shape[-1]
    nb = w1.shape[0]
    c = 2 * bf
    B = 1
    for cand in (8, 6, 4, 3, 2):
        if n % cand == 0:
            B = cand
            break
    body = functools.partial(_stage_kernel, nb=nb, Ho=Ho, Wo=Wo)
    weights = (wdw1, bdw1, wp1, bp1, w21, b21, wdw2, bdw2, w22, b22,
               w1, b1, wdw, bdw, w2, b2, mask)

    def _full(a):
        nd = a.ndim
        return pl.BlockSpec(a.shape, lambda i, _nd=nd: (0,) * _nd)

    flops = n * (2 * 4 * Rh * cin * bf + 2 * Rh * cin * bf + 2 * Rh * bf * bf
                 + 18 * Rh * (cin + bf)
                 + nb * (2 * Rh * c * bf + 2 * Rh * bf * c + 18 * Rh * bf))
    wbytes = sum(int(v.size) * v.dtype.itemsize for v in weights)
    cost = pl.CostEstimate(flops=int(flops), transcendentals=0,
                           bytes_accessed=int(xpar.size * 2 + n * Rh * c * 2
                                              + wbytes))
    return pl.pallas_call(
        body,
        out_shape=jax.ShapeDtypeStruct((n, Rh, c), _BF16),
        grid=(n // B,),
        in_specs=[pl.BlockSpec((B, 2, 2, Rh, cin),
                               lambda i: (i, 0, 0, 0, 0))]
                 + [_full(v) for v in weights],
        out_specs=pl.BlockSpec((B, Rh, c), lambda i: (i, 0, 0)),
        compiler_params=pltpu.CompilerParams(
            dimension_semantics=("parallel",)),
        cost_estimate=cost,
    )(xpar, *weights)


def _stem_conv_kernel(xp_ref, w_ref, b_ref, o_ref):
    B = xp_ref.shape[0]
    Ho, Wo = xp_ref.shape[-2], xp_ref.shape[-1]
    c1 = w_ref.shape[1]
    w = w_ref[...].astype(_F32)
    b = b_ref[...]

    def sel(k):
        return (1, 1) if k == 0 else ((0, 0) if k == 1 else (1, 0))

    taps = [None] * 27
    for ci in range(3):
        p = {(a, bb): xp_ref[:, ci, a, bb].astype(_F32)
             for a in range(2) for bb in range(2)}
        for ky in range(3):
            ra, rs = sel(ky)
            for kx in range(3):
                ca, cs = sel(kx)
                t = p[(ra, ca)]
                if rs:
                    t = jnp.concatenate(
                        [jnp.zeros((B, 1, Wo), _F32), t[:, :-1, :]], axis=1)
                if cs:
                    t = jnp.concatenate(
                        [jnp.zeros((B, Ho, 1), _F32), t[:, :, :-1]], axis=2)
                taps[(ky * 3 + kx) * 3 + ci] = t
    for co in range(c1):
        acc = taps[0] * w[0, co]
        for k in range(1, 27):
            acc = acc + taps[k] * w[k, co]
        o_ref[:, co] = jnp.maximum(acc + b[0, co], 0.0).astype(o_ref.dtype)


def _stem_conv(xp, w, b):
    n = xp.shape[0]
    Ho, Wo = xp.shape[-2], xp.shape[-1]
    c1 = w.shape[1]
    B = 1
    for cand in (4, 3, 2):
        if n % cand == 0:
            B = cand
            break
    cost = pl.CostEstimate(
        flops=int(2 * n * Ho * Wo * 27 * c1), transcendentals=0,
        bytes_accessed=int(xp.size * 2 + n * c1 * Ho * Wo * 2))
    return pl.pallas_call(
        _stem_conv_kernel,
        out_shape=jax.ShapeDtypeStruct((n, c1, Ho, Wo), _BF16),
        grid=(n // B,),
        in_specs=[
            pl.BlockSpec((B, 3, 2, 2, Ho, Wo), lambda i: (i, 0, 0, 0, 0, 0)),
            pl.BlockSpec((27, c1), lambda i: (0, 0)),
            pl.BlockSpec((1, c1), lambda i: (0, 0)),
        ],
        out_specs=pl.BlockSpec((B, c1, Ho, Wo), lambda i: (i, 0, 0, 0)),
        compiler_params=pltpu.CompilerParams(
            dimension_semantics=("parallel",)),
        cost_estimate=cost,
    )(xp, w, b)


def _parity_split(x_nhwc):
    n, h, w, c = x_nhwc.shape
    hh, wh = h // 2, w // 2
    x = x_nhwc.reshape(n, hh, 2, wh, 2, c)
    x = jnp.transpose(x, (0, 2, 4, 1, 3, 5))
    return x.reshape(n, 2, 2, hh * wh, c)


def kernel(x, stem_w, stem_b, st0_s2_wdw1, st0_s2_bdw1, st0_s2_wp1,
           st0_s2_bp1, st0_s2_w21, st0_s2_b21, st0_s2_wdw2, st0_s2_bdw2,
           st0_s2_w22, st0_s2_b22, st0_s1_w1, st0_s1_b1, st0_s1_wdw,
           st0_s1_bdw, st0_s1_w2, st0_s1_b2, st0_s1_mask, st0_out_perm,
           st1_s2_wdw1, st1_s2_bdw1, st1_s2_wp1, st1_s2_bp1, st1_s2_w21,
           st1_s2_b21, st1_s2_wdw2, st1_s2_bdw2, st1_s2_w22, st1_s2_b22,
           st1_s1_w1, st1_s1_b1, st1_s1_wdw, st1_s1_bdw, st1_s1_w2,
           st1_s1_b2, st1_s1_mask, st1_out_perm, st2_s2_wdw1, st2_s2_bdw1,
           st2_s2_wp1, st2_s2_bp1, st2_s2_w21, st2_s2_b21, st2_s2_wdw2,
           st2_s2_bdw2, st2_s2_w22, st2_s2_b22, st2_s1_w1, st2_s1_b1,
           st2_s1_wdw, st2_s1_bdw, st2_s1_w2, st2_s1_b2, st2_s1_mask,
           st2_out_perm):
    xb = x.astype(_BF16)
    n, _, h, w = xb.shape
    ho, wo = h // 2, w // 2
    xp = xb.reshape(n, 3, ho, 2, wo, 2)
    xp = jnp.transpose(xp, (0, 1, 3, 5, 2, 4))
    c1 = stem_w.shape[1]
    conv = _stem_conv(xp, stem_w, stem_b)
    pooled = lax.reduce_window(conv, jnp.array(-jnp.inf, conv.dtype), lax.max,
                               (1, 1, 3, 3), (1, 1, 2, 2),
                               ((0, 0), (0, 0), (1, 1), (1, 1)))
    hh, ww = pooled.shape[-2], pooled.shape[-1]
    fp = pooled.reshape(n, c1, hh // 2, 2, ww // 2, 2)
    feat0 = jnp.transpose(fp, (0, 3, 5, 2, 4, 1)).reshape(
        n, 2, 2, (hh // 2) * (ww // 2), c1)

    stages = [
        (st0_s2_wdw1, st0_s2_bdw1, st0_s2_wp1, st0_s2_bp1, st0_s2_w21,
         st0_s2_b21, st0_s2_wdw2, st0_s2_bdw2, st0_s2_w22, st0_s2_b22,
         st0_s1_w1, st0_s1_b1, st0_s1_wdw, st0_s1_bdw, st0_s1_w2, st0_s1_b2,
         st0_s1_mask, st0_out_perm),
        (st1_s2_wdw1, st1_s2_bdw1, st1_s2_wp1, st1_s2_bp1, st1_s2_w21,
         st1_s2_b21, st1_s2_wdw2, st1_s2_bdw2, st1_s2_w22, st1_s2_b22,
         st1_s1_w1, st1_s1_b1, st1_s1_wdw, st1_s1_bdw, st1_s1_w2, st1_s1_b2,
         st1_s1_mask, st1_out_perm),
        (st2_s2_wdw1, st2_s2_bdw1, st2_s2_wp1, st2_s2_bp1, st2_s2_w21,
         st2_s2_b21, st2_s2_wdw2, st2_s2_bdw2, st2_s2_w22, st2_s2_b22,
         st2_s1_w1, st2_s1_b1, st2_s1_wdw, st2_s1_bdw, st2_s1_w2, st2_s1_b2,
         st2_s1_mask, st2_out_perm),
    ]
    outs = []
    feat = None
    for sidx, sp in enumerate(stages):
        xpar = feat0 if sidx == 0 else _parity_split(feat)
        hh, ww = hh // 2, ww // 2
        xflat = _stage_call(xpar, *sp[:17], Ho=hh, Wo=ww)
        c = xflat.shape[-1]
        feat = xflat.reshape(n, hh, ww, c)
        xo = jnp.take(feat, sp[17], axis=-1)
        outs.append(jnp.transpose(xo, (0, 3, 1, 2)).astype(_F32))
    return tuple(outs)
